# Initial kernel scaffold; baseline (speedup 1.0000x reference)
import jax
import jax.numpy as jnp
from jax.experimental import pallas as pl


def kernel(x1, x2, x3, xf, emb, norm_W, norm_b, xfnorm_W, xfnorm_b, qW, qb, kW, kb, vW, vb, oW, ob):
    raise NotImplementedError("write your pallas kernel here")


# speedup vs baseline: 5.7875x; 5.7875x over previous
import jax
import jax.numpy as jnp
from jax.experimental import pallas as pl

B = 2
T0, T1, T2 = 512, 512, 1024
TT = T0 + T1 + T2
N = 2048
D = 1024
XD = 1024
E = 1024
H = 16
DH = 36
TOPK = 32
NH = N // H

_SEGS = ((0, T0), (T0, T1), (T0 + T1, T2))


def _mono_key(x):
    s = jax.lax.bitcast_convert_type(x, jnp.int32)
    return jnp.where(s >= 0, s, jnp.int32(0x7FFFFFFF) - s)


def _kth_key(keys):
    lo = jnp.min(keys, axis=-1, keepdims=True)
    hi = jnp.max(keys, axis=-1, keepdims=True) + 1

    def body(_, carry):
        lo, hi = carry
        mid = (lo >> 1) + (hi >> 1) + (lo & hi & 1)
        cnt = jnp.sum((keys >= mid).astype(jnp.int32), axis=-1, keepdims=True)
        ge = cnt >= TOPK
        return (jnp.where(ge, mid, lo), jnp.where(ge, hi, mid))

    lo, hi = jax.lax.fori_loop(0, 32, body, (lo, hi))
    return lo


def _mod_kernel(emb_ref, W_ref, b_ref, out_ref):
    s = jax.nn.silu(emb_ref[...])
    W = W_ref[0]
    out_ref[0] = (
        jax.lax.dot_general(s, W, (((1,), (1,)), ((), ())),
                            preferred_element_type=jnp.float32)
        + b_ref[0]
    )


def _norm_kernel(x_ref, mods_ref, o_ref):
    b = pl.program_id(0)
    t = pl.program_id(1)
    seg = jnp.where(t < 2, 0, jnp.where(t < 4, 1, 2))
    x = x_ref[0]
    mu = jnp.mean(x, axis=-1, keepdims=True)
    xc = x - mu
    var = jnp.mean(xc * xc, axis=-1, keepdims=True)
    hn = xc * jax.lax.rsqrt(var + 1e-6)
    m = mods_ref[pl.ds(seg * B + b, 1), :]
    o_ref[0] = hn * (1.0 + m[:, :D]) + m[:, D:]


def _main_kernel(xn_ref, xf_ref, modf_ref, qW_ref, qb_ref, kW_ref, kb_ref,
                 vW_ref, vb_ref, oWt_ref, ob_ref, o1_ref, o2_ref, o3_ref):
    h = pl.program_id(1)

    xfb = xf_ref[0, 0]
    mu = jnp.mean(xfb, axis=-1, keepdims=True)
    xc = xfb - mu
    var = jnp.mean(xc * xc, axis=-1, keepdims=True)
    hn = xc * jax.lax.rsqrt(var + 1e-6)
    m = modf_ref[0]
    xfn = hn * (1.0 + m[:, :XD]) + m[:, XD:]

    kWh = kW_ref[0]
    vWh = vW_ref[0]
    kbh = kb_ref[pl.ds(h, 1), :]
    vbh = vb_ref[pl.ds(h, 1), :]
    K = jax.lax.dot_general(xfn, kWh, (((1,), (1,)), ((), ())),
                            preferred_element_type=jnp.float32) + kbh
    V = jax.lax.dot_general(xfn, vWh, (((1,), (1,)), ((), ())),
                            preferred_element_type=jnp.float32) + vbh

    outs = (o1_ref, o2_ref, o3_ref)
    for i in range(3):
        r0, ri = _SEGS[i]
        qWi = qW_ref[i, 0]
        xni = xn_ref[0, r0:r0 + ri, :]
        qbh = qb_ref[i, pl.ds(h, 1), :]
        Q = jax.lax.dot_general(xni, qWi, (((1,), (1,)), ((), ())),
                                preferred_element_type=jnp.float32) + qbh
        logits = jax.lax.dot_general(Q, K, (((1,), (1,)), ((), ())),
                                     preferred_element_type=jnp.float32) / 6.0

        keys = _mono_key(logits)
        thr = _kth_key(keys)
        rmax = jnp.max(logits, axis=-1, keepdims=True)
        w = jnp.where(keys >= thr, jnp.exp(logits - rmax), 0.0)
        wn = w * (1.0 / jnp.sum(w, axis=-1, keepdims=True))

        ov = jax.lax.dot_general(wn, V, (((1,), (0,)), ((), ())),
                                 preferred_element_type=jnp.float32)
        oWti = oWt_ref[i, 0]
        contrib = jax.lax.dot_general(ov, oWti, (((1,), (0,)), ((), ())),
                                      preferred_element_type=jnp.float32)
        obi = ob_ref[pl.ds(i, 1), :]

        @pl.when(h == 0)
        def _():
            outs[i][0] = contrib + obi

        @pl.when(h != 0)
        def _():
            outs[i][0] += contrib


def kernel(x1, x2, x3, xf, emb, norm_W, norm_b, xfnorm_W, xfnorm_b,
           qW, qb, kW, kb, vW, vb, oW, ob):
    f32 = jnp.float32
    Wall = jnp.concatenate([norm_W, xfnorm_W[None]], axis=0)
    ball = jnp.concatenate([norm_b, xfnorm_b[None]], axis=0)
    x_all = jnp.concatenate([x1, x2, x3], axis=1)
    xf4 = xf.reshape(B, H, NH, XD)
    qW4 = qW.reshape(3, H, DH, D)
    qb3 = qb.reshape(3, H, DH)
    oWt = oW.reshape(3, D, H, DH).transpose(0, 2, 3, 1)

    mods = pl.pallas_call(
        _mod_kernel,
        grid=(4,),
        in_specs=[
            pl.BlockSpec((B, E), lambda i: (0, 0)),
            pl.BlockSpec((1, 2 * D, E), lambda i: (i, 0, 0)),
            pl.BlockSpec((1, 1, 2 * D), lambda i: (i, 0, 0)),
        ],
        out_specs=pl.BlockSpec((1, B, 2 * D), lambda i: (i, 0, 0)),
        out_shape=jax.ShapeDtypeStruct((4, B, 2 * D), f32),
    )(emb, Wall, ball.reshape(4, 1, 2 * D))
    mods8 = mods.reshape(4 * B, 2 * D)
    mods83 = mods.reshape(4 * B, 1, 2 * D)

    TBLK = 256
    xn = pl.pallas_call(
        _norm_kernel,
        grid=(B, TT // TBLK),
        in_specs=[
            pl.BlockSpec((1, TBLK, D), lambda b, t: (b, t, 0)),
            pl.BlockSpec((4 * B, 2 * D), lambda b, t: (0, 0)),
        ],
        out_specs=pl.BlockSpec((1, TBLK, D), lambda b, t: (b, t, 0)),
        out_shape=jax.ShapeDtypeStruct((B, TT, D), f32),
    )(x_all, mods8)

    out1, out2, out3 = pl.pallas_call(
        _main_kernel,
        grid=(B, H),
        in_specs=[
            pl.BlockSpec((1, TT, D), lambda b, h: (b, 0, 0)),
            pl.BlockSpec((1, 1, NH, XD), lambda b, h: (b, h, 0, 0)),
            pl.BlockSpec((1, 1, 2 * XD), lambda b, h: (3 * B + b, 0, 0)),
            pl.BlockSpec((3, 1, DH, D), lambda b, h: (0, h, 0, 0)),
            pl.BlockSpec((3, H, DH), lambda b, h: (0, 0, 0)),
            pl.BlockSpec((1, DH, XD), lambda b, h: (h, 0, 0)),
            pl.BlockSpec((H, DH), lambda b, h: (0, 0)),
            pl.BlockSpec((1, DH, XD), lambda b, h: (h, 0, 0)),
            pl.BlockSpec((H, DH), lambda b, h: (0, 0)),
            pl.BlockSpec((3, 1, DH, D), lambda b, h: (0, h, 0, 0)),
            pl.BlockSpec((3, D), lambda b, h: (0, 0)),
        ],
        out_specs=[
            pl.BlockSpec((1, T0, D), lambda b, h: (b, 0, 0)),
            pl.BlockSpec((1, T1, D), lambda b, h: (b, 0, 0)),
            pl.BlockSpec((1, T2, D), lambda b, h: (b, 0, 0)),
        ],
        out_shape=[
            jax.ShapeDtypeStruct((B, T0, D), f32),
            jax.ShapeDtypeStruct((B, T1, D), f32),
            jax.ShapeDtypeStruct((B, T2, D), f32),
        ],
    )(xn, xf4, mods83, qW4, qb3, kW, kb, vW, vb, oWt, ob)

    return (out1, out2, out3)


# speedup vs baseline: 22.0595x; 3.8116x over previous
import jax
import jax.numpy as jnp
from jax.experimental import pallas as pl
from jax.experimental.pallas import tpu as pltpu

B = 2
T0, T1, T2 = 512, 512, 1024
TT = T0 + T1 + T2
N = 2048
D = 1024
XD = 1024
E = 1024
H = 16
DH = 36
TOPK = 32
NH = N // H
DHP = 40

_SEGS = ((0, T0), (T0, T1), (T0 + T1, T2))


def _mono_key(x):
    s = jax.lax.bitcast_convert_type(x, jnp.int32)
    return jnp.where(s >= 0, s, jnp.int32(0x7FFFFFFF) - s)


def _kth_key_t(keys):
    lo = jnp.min(keys, axis=0, keepdims=True)
    hi = jnp.max(keys, axis=0, keepdims=True) + 1

    def body(_, carry):
        lo, hi = carry
        mid = (lo >> 1) + (hi >> 1) + (lo & hi & 1)
        cnt = jnp.sum((keys >= mid).astype(jnp.float32), axis=0, keepdims=True)
        ge = cnt >= float(TOPK)
        return (jnp.where(ge, mid, lo), jnp.where(ge, hi, mid))

    lo, hi = jax.lax.fori_loop(0, 32, body, (lo, hi))
    return lo


def _mod_kernel(emb_ref, W_ref, b_ref, out_ref):
    s = jax.nn.silu(emb_ref[...])
    W = W_ref[0]
    out_ref[0] = (
        jax.lax.dot_general(s, W, (((1,), (1,)), ((), ())),
                            preferred_element_type=jnp.float32)
        + b_ref[0]
    )


def _norm_kernel(x_ref, mods_ref, o_ref):
    b = pl.program_id(0)
    t = pl.program_id(1)
    seg = jnp.where(t < 2, 0, jnp.where(t < 4, 1, 2))
    x = x_ref[0]
    mu = jnp.mean(x, axis=-1, keepdims=True)
    xc = x - mu
    var = jnp.mean(xc * xc, axis=-1, keepdims=True)
    hn = xc * jax.lax.rsqrt(var + 1e-6)
    m = mods_ref[pl.ds(seg * B + b, 1), :]
    o_ref[0] = hn * (1.0 + m[:, :D]) + m[:, D:]


def _main_kernel(xn_ref, xf_ref, modf_ref, qW_ref, qb_ref, kW_ref, kb_ref,
                 vW_ref, vb_ref, oWT_ref, ob_ref, o1_ref, o2_ref, o3_ref,
                 ovt_ref):
    h = pl.program_id(1)

    xfb = xf_ref[0, 0]
    mu = jnp.mean(xfb, axis=-1, keepdims=True)
    xc = xfb - mu
    var = jnp.mean(xc * xc, axis=-1, keepdims=True)
    hn = xc * jax.lax.rsqrt(var + 1e-6)
    m = modf_ref[0]
    xfn = hn * (1.0 + m[:, :XD]) + m[:, XD:]

    kWh = kW_ref[0]
    vWh = vW_ref[0]
    kbh = kb_ref[pl.ds(h, 1), :]
    vbh = vb_ref[pl.ds(h, 1), :]
    K = jax.lax.dot_general(xfn, kWh, (((1,), (1,)), ((), ())),
                            preferred_element_type=jnp.float32) + kbh
    V = jax.lax.dot_general(xfn, vWh, (((1,), (1,)), ((), ())),
                            preferred_element_type=jnp.float32) + vbh

    for i in range(3):
        r0, ri = _SEGS[i]
        qWi = qW_ref[i, 0]
        xni = xn_ref[0, r0:r0 + ri, :]
        qbh = qb_ref[i, pl.ds(h, 1), :]
        Q = jax.lax.dot_general(xni, qWi, (((1,), (1,)), ((), ())),
                                preferred_element_type=jnp.float32) + qbh
        logitsT = jax.lax.dot_general(K, Q, (((1,), (1,)), ((), ())),
                                      preferred_element_type=jnp.float32) / 6.0

        keys = _mono_key(logitsT)
        thr = _kth_key_t(keys)
        rmax = jnp.max(logitsT, axis=0, keepdims=True)
        w = jnp.where(keys >= thr, jnp.exp(logitsT - rmax), 0.0)
        wnT = w * (1.0 / jnp.sum(w, axis=0, keepdims=True))

        ovT = jax.lax.dot_general(V, wnT, (((0,), (0,)), ((), ())),
                                  preferred_element_type=jnp.float32)
        ovTp = jnp.concatenate(
            [ovT, jnp.zeros((DHP - DH, ri), jnp.float32)], axis=0)
        ovt_ref[pl.ds(h * DHP, DHP), r0:r0 + ri] = ovTp

    @pl.when(h == H - 1)
    def _():
        outs = (o1_ref, o2_ref, o3_ref)
        for i in range(3):
            r0, ri = _SEGS[i]
            ovi = ovt_ref[:, r0:r0 + ri].T
            oWTi = oWT_ref[i]
            obi = ob_ref[pl.ds(i, 1), :]
            outs[i][0] = jax.lax.dot_general(
                ovi, oWTi, (((1,), (0,)), ((), ())),
                preferred_element_type=jnp.float32) + obi


def kernel(x1, x2, x3, xf, emb, norm_W, norm_b, xfnorm_W, xfnorm_b,
           qW, qb, kW, kb, vW, vb, oW, ob):
    f32 = jnp.float32
    Wall = jnp.concatenate([norm_W, xfnorm_W[None]], axis=0)
    ball = jnp.concatenate([norm_b, xfnorm_b[None]], axis=0)
    x_all = jnp.concatenate([x1, x2, x3], axis=1)
    xf4 = xf.reshape(B, H, NH, XD)
    qW4 = qW.reshape(3, H, DH, D)
    qb3 = qb.reshape(3, H, DH)
    oWT = jnp.pad(oW.transpose(0, 2, 1).reshape(3, H, DH, D),
                  ((0, 0), (0, 0), (0, DHP - DH), (0, 0))).reshape(3, H * DHP, D)

    mods = pl.pallas_call(
        _mod_kernel,
        grid=(4,),
        in_specs=[
            pl.BlockSpec((B, E), lambda i: (0, 0)),
            pl.BlockSpec((1, 2 * D, E), lambda i: (i, 0, 0)),
            pl.BlockSpec((1, 1, 2 * D), lambda i: (i, 0, 0)),
        ],
        out_specs=pl.BlockSpec((1, B, 2 * D), lambda i: (i, 0, 0)),
        out_shape=jax.ShapeDtypeStruct((4, B, 2 * D), f32),
    )(emb, Wall, ball.reshape(4, 1, 2 * D))
    mods8 = mods.reshape(4 * B, 2 * D)
    mods83 = mods.reshape(4 * B, 1, 2 * D)

    TBLK = 256
    xn = pl.pallas_call(
        _norm_kernel,
        grid=(B, TT // TBLK),
        in_specs=[
            pl.BlockSpec((1, TBLK, D), lambda b, t: (b, t, 0)),
            pl.BlockSpec((4 * B, 2 * D), lambda b, t: (0, 0)),
        ],
        out_specs=pl.BlockSpec((1, TBLK, D), lambda b, t: (b, t, 0)),
        out_shape=jax.ShapeDtypeStruct((B, TT, D), f32),
    )(x_all, mods8)

    out1, out2, out3 = pl.pallas_call(
        _main_kernel,
        grid=(B, H),
        in_specs=[
            pl.BlockSpec((1, TT, D), lambda b, h: (b, 0, 0)),
            pl.BlockSpec((1, 1, NH, XD), lambda b, h: (b, h, 0, 0)),
            pl.BlockSpec((1, 1, 2 * XD), lambda b, h: (3 * B + b, 0, 0)),
            pl.BlockSpec((3, 1, DH, D), lambda b, h: (0, h, 0, 0)),
            pl.BlockSpec((3, H, DH), lambda b, h: (0, 0, 0)),
            pl.BlockSpec((1, DH, XD), lambda b, h: (h, 0, 0)),
            pl.BlockSpec((H, DH), lambda b, h: (0, 0)),
            pl.BlockSpec((1, DH, XD), lambda b, h: (h, 0, 0)),
            pl.BlockSpec((H, DH), lambda b, h: (0, 0)),
            pl.BlockSpec((3, H * DHP, D), lambda b, h: (0, 0, 0)),
            pl.BlockSpec((3, D), lambda b, h: (0, 0)),
        ],
        scratch_shapes=[pltpu.VMEM((H * DHP, TT), f32)],
        out_specs=[
            pl.BlockSpec((1, T0, D), lambda b, h: (b, 0, 0)),
            pl.BlockSpec((1, T1, D), lambda b, h: (b, 0, 0)),
            pl.BlockSpec((1, T2, D), lambda b, h: (b, 0, 0)),
        ],
        out_shape=[
            jax.ShapeDtypeStruct((B, T0, D), f32),
            jax.ShapeDtypeStruct((B, T1, D), f32),
            jax.ShapeDtypeStruct((B, T2, D), f32),
        ],
    )(xn, xf4, mods83, qW4, qb3, kW, kb, vW, vb, oWT, ob)

    return (out1, out2, out3)
